# baseline (speedup 1.0000x reference)
import functools

import jax
import jax.numpy as jnp
from jax import lax
from jax.experimental import pallas as pl
from jax.experimental.pallas import tpu as pltpu
from jax.experimental.pallas import tpu_sc as plsc

NC, NS = 2, 16
NW = NC * NS
LANES = 16
PACK = 128


def _sc_scores(ctx_row, ctx_off, cen_row, cen_off, neg_row, neg_off,
               in2, out2, B, CTX, NEG, D):
    BW = B // NW
    C = 16
    NIT = BW // C
    NKC = D // LANES

    mesh = plsc.VectorSubcoreMesh(core_axis_name="c", subcore_axis_name="s")

    @functools.partial(
        pl.kernel,
        out_type=(
            jax.ShapeDtypeStruct((B,), jnp.float32),
            jax.ShapeDtypeStruct((B * NEG,), jnp.float32),
        ),
        mesh=mesh,
        compiler_params=pltpu.CompilerParams(needs_layout_passes=False),
        scratch_types=[
            pltpu.VMEM((BW * CTX,), jnp.int32),
            pltpu.VMEM((BW,), jnp.int32),
            pltpu.VMEM((BW * NEG,), jnp.int32),
            pltpu.VMEM((BW * CTX + LANES,), jnp.int32),
            pltpu.VMEM((BW,), jnp.int32),
            pltpu.VMEM((BW * NEG + 2 * LANES,), jnp.int32),
            pltpu.VMEM((C * CTX, PACK), jnp.float32),
            pltpu.VMEM((C, PACK), jnp.float32),
            pltpu.VMEM((C * NEG, PACK), jnp.float32),
            pltpu.VMEM((C * LANES,), jnp.float32),
            pltpu.VMEM((C * NEG * LANES,), jnp.float32),
            pltpu.VMEM((BW,), jnp.float32),
            pltpu.VMEM((BW * NEG,), jnp.float32),
            pltpu.SemaphoreType.DMA,
        ],
    )
    def score_kernel(ctxr_hbm, ctxo_hbm, cenr_hbm, ceno_hbm, negr_hbm,
                     nego_hbm, ine_hbm, oute_hbm,
                     pos_o_hbm, neg_o_hbm,
                     ctx_idx, cen_idx, neg_idx, ctx_par, cen_par, neg_par,
                     ctx_rows, pos_rows, neg_rows,
                     stage_pos, stage_neg, pos_buf, neg_buf, sem):
        wid = lax.axis_index("s") * NC + lax.axis_index("c")
        pltpu.sync_copy(ctxr_hbm.at[pl.ds(wid * BW * CTX, BW * CTX)], ctx_idx)
        pltpu.sync_copy(cenr_hbm.at[pl.ds(wid * BW, BW)], cen_idx)
        pltpu.sync_copy(negr_hbm.at[pl.ds(wid * BW * NEG, BW * NEG)], neg_idx)
        pltpu.sync_copy(ctxo_hbm.at[pl.ds(wid * BW * CTX, BW * CTX)],
                        ctx_par.at[pl.ds(0, BW * CTX)])
        pltpu.sync_copy(ceno_hbm.at[pl.ds(wid * BW, BW)], cen_par)
        pltpu.sync_copy(nego_hbm.at[pl.ds(wid * BW * NEG, BW * NEG)],
                        neg_par.at[pl.ds(0, BW * NEG)])

        def colsum(stage, r0):
            base = lax.iota(jnp.int32, 16) * LANES + (r0 * LANES)
            acc = plsc.load_gather(stage, [base])
            for c in range(1, LANES):
                acc = acc + plsc.load_gather(stage, [base + c])
            return acc

        def body(i, carry):
            dmas = []
            nctx = C * CTX
            for h in range(2):
                dmas.append(pltpu.async_copy(
                    ine_hbm.at[ctx_idx.at[pl.ds(i * nctx + h * (nctx // 2),
                                                nctx // 2)]],
                    ctx_rows.at[pl.ds(h * (nctx // 2), nctx // 2)], sem))
            dmas.append(pltpu.async_copy(
                oute_hbm.at[cen_idx.at[pl.ds(i * C, C)]], pos_rows, sem))
            nneg = C * NEG
            for h in range(4):
                dmas.append(pltpu.async_copy(
                    oute_hbm.at[neg_idx.at[pl.ds(i * nneg + h * (nneg // 4),
                                                 nneg // 4)]],
                    neg_rows.at[pl.ds(h * (nneg // 4), nneg // 4)], sem))
            for d in dmas:
                d.wait()

            cen_pv = cen_par[pl.ds(i * C, LANES)]
            for b in range(C):
                ctx_pv = ctx_par[pl.ds(i * nctx + b * CTX, LANES)]
                o = ctx_pv[0]
                m = [ctx_rows[b * CTX, pl.ds(o + k * LANES, LANES)]
                     for k in range(NKC)]
                for c in range(1, CTX):
                    row = b * CTX + c
                    o = ctx_pv[c]
                    m = [m[k] + ctx_rows[row, pl.ds(o + k * LANES, LANES)]
                         for k in range(NKC)]
                m = [mk * (1.0 / CTX) for mk in m]

                o = cen_pv[b]
                p = m[0] * pos_rows[b, pl.ds(o, LANES)]
                for k in range(1, NKC):
                    p = p + m[k] * pos_rows[b, pl.ds(o + k * LANES, LANES)]
                stage_pos[pl.ds(b * LANES, LANES)] = p

                neg_pv = [neg_par[pl.ds(i * nneg + b * NEG, LANES)],
                          neg_par[pl.ds(i * nneg + b * NEG + LANES, LANES)]]
                for n in range(NEG):
                    row = b * NEG + n
                    o = neg_pv[n // LANES][n % LANES]
                    q = m[0] * neg_rows[row, pl.ds(o, LANES)]
                    for k in range(1, NKC):
                        q = q + m[k] * neg_rows[row, pl.ds(o + k * LANES,
                                                           LANES)]
                    stage_neg[pl.ds(row * LANES, LANES)] = q

            pos_buf[pl.ds(i * C, C)] = colsum(stage_pos, 0)
            for g in range(C * NEG // LANES):
                neg_buf[pl.ds(i * (C * NEG) + g * LANES, LANES)] = (
                    colsum(stage_neg, g * LANES))
            return carry

        lax.fori_loop(0, NIT, body, 0)
        pltpu.sync_copy(pos_buf, pos_o_hbm.at[pl.ds(wid * BW, BW)])
        pltpu.sync_copy(neg_buf, neg_o_hbm.at[pl.ds(wid * BW * NEG, BW * NEG)])

    return score_kernel(ctx_row, ctx_off, cen_row, cen_off, neg_row, neg_off,
                        in2, out2)


def _loss_from_scores(pos_score, neg_score_flat, B):
    pos2 = pos_score.reshape(-1, 128)
    neg2 = neg_score_flat.reshape(-1, 128)

    def body(p_ref, n_ref, o_ref):
        def neg_softplus(x):
            return jnp.minimum(x, 0.0) - jnp.log(1.0 + jnp.exp(-jnp.abs(x)))

        total = jnp.sum(neg_softplus(p_ref[...]))
        total = total + jnp.sum(neg_softplus(-n_ref[...]))
        o_ref[0, 0] = -total / B

    out = pl.pallas_call(
        body,
        out_shape=jax.ShapeDtypeStruct((1, 1), jnp.float32),
        out_specs=pl.BlockSpec(memory_space=pltpu.SMEM),
    )(pos2, neg2)
    return out[0, 0]


def kernel(context_words, center_word, neg_words, in_embed, out_embed):
    B, CTX = context_words.shape
    NEG = neg_words.shape[1]
    V, D = in_embed.shape
    in2 = in_embed.reshape(V * D // PACK, PACK)
    out2 = out_embed.reshape(V * D // PACK, PACK)

    def split(idx):
        flat = idx.reshape(-1)
        return flat >> 1, (flat & 1) * D

    ctx_row, ctx_off = split(context_words)
    cen_row, cen_off = split(center_word)
    neg_row, neg_off = split(neg_words)
    pos_score, neg_score = _sc_scores(
        ctx_row, ctx_off, cen_row, cen_off, neg_row, neg_off,
        in2, out2, B, CTX, NEG, D)
    return _loss_from_scores(pos_score, neg_score, B)
